# Initial kernel scaffold; baseline (speedup 1.0000x reference)
import jax
import jax.numpy as jnp
from jax.experimental import pallas as pl


def kernel(tokens, table):
    raise NotImplementedError("write your pallas kernel here")


# speedup vs baseline: 4.9068x; 4.9068x over previous
import functools
import math

import jax
import jax.numpy as jnp
from jax import lax
from jax.experimental import pallas as pl
from jax.experimental.pallas import tpu as pltpu
import jax.experimental.pallas.tpu_sc as plsc

VOCAB = 1000000
EMBED = 32
PAD_IDX = 0
SCALE = math.sqrt(EMBED)

NUM_CORES = 2
NUM_SUBCORES = 16
NW = NUM_CORES * NUM_SUBCORES

B_TOTAL = 16384 * 200
B_PER_W = B_TOTAL // NW
CHUNK = 128
NCHUNK = B_PER_W // CHUNK
RING = 4
LOOKAHEAD = 2


def _body(tok_ref, tab_ref, out_ref, idx_v, rows_v, gsems, osems):
    cid = lax.axis_index("c")
    sid = lax.axis_index("s")
    wid = sid * NUM_CORES + cid

    pltpu.sync_copy(tok_ref.at[wid], idx_v)

    def start_gather(h, slot):
        pltpu.make_async_copy(
            tab_ref.at[idx_v.at[h]], rows_v.at[slot], gsems[slot]
        ).start()

    def wait_gather(slot):
        pltpu.make_async_copy(
            tab_ref.at[idx_v.at[0]], rows_v.at[slot], gsems[slot]
        ).wait()

    def start_writeback(g, slot):
        pltpu.make_async_copy(
            rows_v.at[slot], out_ref.at[wid, g], osems[slot]
        ).start()

    def wait_writeback(slot):
        pltpu.make_async_copy(
            rows_v.at[slot], out_ref.at[wid, 0], osems[slot]
        ).wait()

    for h in range(LOOKAHEAD):
        start_gather(h, h)

    def super_body(it, carry):
        g0 = it * RING
        for b in range(RING):
            g = g0 + b
            h = g + LOOKAHEAD
            hb = (b + LOOKAHEAD) % RING

            @pl.when(jnp.logical_and(h < NCHUNK, h >= RING))
            def _():
                wait_writeback(hb)

            @pl.when(h < NCHUNK)
            def _():
                start_gather(h, hb)

            wait_gather(b)

            def group_fix(j, c):
                ivec = idx_v[g, pl.ds(j * 16, 16)]
                scv = jnp.where(ivec == PAD_IDX, 0.0, SCALE).astype(jnp.float32)
                for r in range(16):
                    i = j * 16 + r
                    sc = scv[r]
                    rows_v[b, i, pl.ds(0, 16)] = rows_v[b, i, pl.ds(0, 16)] * sc
                    rows_v[b, i, pl.ds(16, 16)] = (
                        rows_v[b, i, pl.ds(16, 16)] * sc
                    )
                return c

            lax.fori_loop(0, CHUNK // 16, group_fix, 0)

            start_writeback(g, b)
        return carry

    lax.fori_loop(0, NCHUNK // RING, super_body, 0)

    for b in range(RING):
        wait_writeback(b)


@functools.partial(jax.jit, static_argnames=())
def kernel(tokens, table):
    tok = tokens.reshape(-1).astype(jnp.int32).reshape(NW, NCHUNK, CHUNK)
    mesh = plsc.VectorSubcoreMesh(core_axis_name="c", subcore_axis_name="s")
    out = pl.kernel(
        _body,
        out_type=jax.ShapeDtypeStruct((NW, NCHUNK, CHUNK, EMBED), jnp.float32),
        mesh=mesh,
        scratch_types=[
            pltpu.VMEM((NCHUNK, CHUNK), jnp.int32),
            pltpu.VMEM((RING, CHUNK, EMBED), jnp.float32),
            [pltpu.SemaphoreType.DMA] * RING,
            [pltpu.SemaphoreType.DMA] * RING,
        ],
        compiler_params=pltpu.CompilerParams(use_tc_tiling_on_sc=False),
        name="token_embedding_sc",
    )(tok, table)
    return out.reshape(tokens.shape[0], tokens.shape[1], EMBED)


# speedup vs baseline: 5.0031x; 1.0196x over previous
import functools
import math

import jax
import jax.numpy as jnp
from jax import lax
from jax.experimental import pallas as pl
from jax.experimental.pallas import tpu as pltpu
import jax.experimental.pallas.tpu_sc as plsc

VOCAB = 1000000
EMBED = 32
PAD_IDX = 0
SCALE = math.sqrt(EMBED)

NUM_CORES = 2
NUM_SUBCORES = 16
NW = NUM_CORES * NUM_SUBCORES

ROWS = 16384
COLS = 200
R_PER_W = ROWS // NW
RING = 4
LOOKAHEAD = 2
SPLIT = 128


def _body(tok_ref, tab_ref, out_ref, idx_v, rows_v, gsems, osems):
    cid = lax.axis_index("c")
    sid = lax.axis_index("s")
    wid = sid * NUM_CORES + cid
    row0 = wid * R_PER_W

    pltpu.sync_copy(tok_ref.at[pl.ds(row0, R_PER_W), :], idx_v)

    def start_gather(g, slot):
        pltpu.make_async_copy(
            tab_ref.at[idx_v.at[g, pl.ds(0, SPLIT)]],
            rows_v.at[slot, pl.ds(0, SPLIT)],
            gsems[slot],
        ).start()
        pltpu.make_async_copy(
            tab_ref.at[idx_v.at[g, pl.ds(SPLIT, COLS - SPLIT)]],
            rows_v.at[slot, pl.ds(SPLIT, COLS - SPLIT)],
            gsems[slot],
        ).start()

    def wait_gather(slot):
        pltpu.make_async_copy(
            tab_ref.at[idx_v.at[0, pl.ds(0, SPLIT)]],
            rows_v.at[slot, pl.ds(0, SPLIT)],
            gsems[slot],
        ).wait()
        pltpu.make_async_copy(
            tab_ref.at[idx_v.at[0, pl.ds(SPLIT, COLS - SPLIT)]],
            rows_v.at[slot, pl.ds(SPLIT, COLS - SPLIT)],
            gsems[slot],
        ).wait()

    def start_writeback(g, slot):
        pltpu.make_async_copy(
            rows_v.at[slot], out_ref.at[row0 + g], osems[slot]
        ).start()

    def wait_writeback(slot):
        pltpu.make_async_copy(
            rows_v.at[slot], out_ref.at[row0], osems[slot]
        ).wait()

    for g in range(LOOKAHEAD):
        start_gather(g, g)

    def fix16(g, slot, base, lane0):
        ivec = idx_v[g, pl.ds(base, 16)]
        scv = jnp.where(ivec == PAD_IDX, 0.0, SCALE).astype(jnp.float32)
        for r in range(lane0, 16):
            i = base + r
            sc = scv[r]
            rows_v[slot, i, pl.ds(0, 16)] = rows_v[slot, i, pl.ds(0, 16)] * sc
            rows_v[slot, i, pl.ds(16, 16)] = (
                rows_v[slot, i, pl.ds(16, 16)] * sc
            )

    def super_body(it, carry):
        g0 = it * RING
        for b in range(RING):
            g = g0 + b
            h = g + LOOKAHEAD
            hb = (b + LOOKAHEAD) % RING

            @pl.when(jnp.logical_and(h < R_PER_W, h >= RING))
            def _():
                wait_writeback(hb)

            @pl.when(h < R_PER_W)
            def _():
                start_gather(h, hb)

            wait_gather(b)

            def group_fix(j, c):
                fix16(g, b, j * 16, 0)
                return c

            lax.fori_loop(0, (COLS - 8) // 16, group_fix, 0)
            fix16(g, b, COLS - 16, 8)

            start_writeback(g, b)
        return carry

    lax.fori_loop(0, R_PER_W // RING, super_body, 0)

    for b in range(RING):
        wait_writeback(b)


@jax.jit
def kernel(tokens, table):
    tok = tokens.astype(jnp.int32)
    mesh = plsc.VectorSubcoreMesh(core_axis_name="c", subcore_axis_name="s")
    return pl.kernel(
        _body,
        out_type=jax.ShapeDtypeStruct((ROWS, COLS, EMBED), jnp.float32),
        mesh=mesh,
        scratch_types=[
            pltpu.VMEM((R_PER_W, COLS), jnp.int32),
            pltpu.VMEM((RING, COLS, EMBED), jnp.float32),
            [pltpu.SemaphoreType.DMA] * RING,
            [pltpu.SemaphoreType.DMA] * RING,
        ],
        compiler_params=pltpu.CompilerParams(use_tc_tiling_on_sc=False),
        name="token_embedding_sc",
    )(tok, table)
